# baseline (speedup 1.0000x reference)
import functools

import jax
import jax.numpy as jnp
from jax import lax
from jax.experimental import pallas as pl
from jax.experimental.pallas import tpu as pltpu
from jax.experimental.pallas import tpu_sc as plsc

BATCH = 16384
SEQ = 10
EMBED_DIM = 64

_B = BATCH * SEQ
_NC = 2
_NS = 16
_NW = _NC * _NS
_B_PER_W = _B // _NW
_CHUNK = 512
_N_CHUNKS = _B_PER_W // _CHUNK
_NBUF = 2


@functools.partial(
    pl.kernel,
    mesh=plsc.VectorSubcoreMesh(core_axis_name="c", subcore_axis_name="s"),
    compiler_params=pltpu.CompilerParams(use_tc_tiling_on_sc=False),
    out_type=jax.ShapeDtypeStruct((_B, EMBED_DIM), jnp.float32),
    scratch_types=[
        pltpu.VMEM((_N_CHUNKS, _CHUNK), jnp.int32),
        pltpu.VMEM((_NBUF, _CHUNK, EMBED_DIM), jnp.float32),
        pltpu.SemaphoreType.DMA((_NBUF,)),
        pltpu.SemaphoreType.DMA((_NBUF,)),
    ],
)
def _embedding_gather(idx_hbm, table_hbm, out_hbm, idx_v, rows_v, gsem, wsem):
    wid = lax.axis_index("s") * _NC + lax.axis_index("c")
    base = wid * _B_PER_W

    pltpu.sync_copy(idx_hbm.at[pl.ds(wid * _N_CHUNKS, _N_CHUNKS)], idx_v)

    def gather(i):
        b = i % _NBUF
        return pltpu.async_copy(table_hbm.at[idx_v.at[i]], rows_v.at[b],
                                gsem.at[b])

    def write(i):
        b = i % _NBUF
        return pltpu.async_copy(rows_v.at[b],
                                out_hbm.at[pl.ds(base + i * _CHUNK, _CHUNK)],
                                wsem.at[b])

    gathers = [None] * _N_CHUNKS
    writes = [None] * _N_CHUNKS
    gathers[0] = gather(0)
    for i in range(_N_CHUNKS):
        gathers[i].wait()
        if i + 1 < _N_CHUNKS:
            if i >= 1:
                writes[i - 1].wait()
            gathers[i + 1] = gather(i + 1)
        writes[i] = write(i)
    writes[_N_CHUNKS - 2].wait()
    writes[_N_CHUNKS - 1].wait()


def kernel(inputs, table):
    idx = inputs.reshape(_NW * _N_CHUNKS, _CHUNK)
    out = _embedding_gather(idx, table)
    return out.reshape(BATCH, SEQ, EMBED_DIM)


# speedup vs baseline: 1.0000x; 1.0000x over previous
import functools

import jax
import jax.numpy as jnp
from jax import lax
from jax.experimental import pallas as pl
from jax.experimental.pallas import tpu as pltpu
from jax.experimental.pallas import tpu_sc as plsc

BATCH = 16384
SEQ = 10
EMBED_DIM = 64

_B = BATCH * SEQ
_NC = 2
_NS = 16
_NW = _NC * _NS
_B_PER_W = _B // _NW
_CB = 64
_CHUNK = _CB * SEQ
_N_CHUNKS = _B_PER_W // _CHUNK
_NBUF = 2


@functools.partial(
    pl.kernel,
    mesh=plsc.VectorSubcoreMesh(core_axis_name="c", subcore_axis_name="s"),
    compiler_params=pltpu.CompilerParams(use_tc_tiling_on_sc=False),
    out_type=jax.ShapeDtypeStruct((BATCH, SEQ, EMBED_DIM), jnp.float32),
    scratch_types=[
        pltpu.VMEM((_N_CHUNKS, _CHUNK), jnp.int32),
        pltpu.VMEM((_NBUF, _CHUNK, EMBED_DIM), jnp.float32),
        pltpu.SemaphoreType.DMA((_NBUF,)),
        pltpu.SemaphoreType.DMA((_NBUF,)),
    ],
)
def _embedding_gather(idx_hbm, table_hbm, out_hbm, idx_v, rows_v, gsem, wsem):
    wid = lax.axis_index("s") * _NC + lax.axis_index("c")
    base_batch = wid * (_B_PER_W // SEQ)

    pltpu.sync_copy(idx_hbm.at[pl.ds(wid * _N_CHUNKS, _N_CHUNKS)], idx_v)

    def start_gather(j, b):
        return pltpu.async_copy(table_hbm.at[idx_v.at[j]], rows_v.at[b],
                                gsem.at[b])

    def issue_writes(j, b):
        for t in range(_CB):
            pltpu.async_copy(
                rows_v.at[b, pl.ds(t * SEQ, SEQ)],
                out_hbm.at[base_batch + j * _CB + t],
                wsem.at[b])

    def wait_writes(b):
        pltpu.make_async_copy(rows_v.at[b], table_hbm.at[pl.ds(0, _CHUNK)],
                              wsem.at[b]).wait()

    g0 = start_gather(0, 0)
    g1 = start_gather(1, 1)
    g0.wait()
    issue_writes(0, 0)
    g1.wait()
    issue_writes(1, 1)

    @pl.loop(1, _N_CHUNKS // _NBUF)
    def _chunk_pair(g):
        for b in range(_NBUF):
            j = _NBUF * g + b
            wait_writes(b)
            start_gather(j, b).wait()
            issue_writes(j, b)

    for b in range(_NBUF):
        wait_writes(b)


def kernel(inputs, table):
    idx = inputs.reshape(_NW * _N_CHUNKS, _CHUNK)
    return _embedding_gather(idx, table)
